# baseline (speedup 1.0000x reference)
import functools

import jax
import jax.numpy as jnp
from jax.experimental import pallas as pl
from jax.experimental.pallas import tpu as pltpu

K_NEIGHBORS = 50
DELTA = 1e-3


def _enc_body(x_ref, w1_ref, b1_ref, w2_ref, b2_ref, key_ref):
    h = jax.lax.dot_general(
        x_ref[...], w1_ref[...], (((1,), (0,)), ((), ())),
        preferred_element_type=jnp.float32)
    h = jnp.maximum(h + b1_ref[...], 0.0)
    key_ref[...] = jax.lax.dot_general(
        h, w2_ref[...], (((1,), (0,)), ((), ())),
        preferred_element_type=jnp.float32) + b2_ref[...]


def _encoder(x, W1, b1, W2, b2):
    B = x.shape[0]
    D = W2.shape[1]
    return pl.pallas_call(
        _enc_body,
        out_shape=jax.ShapeDtypeStruct((B, D), jnp.float32),
    )(x, W1, b1.reshape(1, -1), W2, b2.reshape(1, -1))


def _dist_body(q_ref, k_ref, d2_ref):
    q = q_ref[...]
    k = k_ref[0]
    qk = jax.lax.dot_general(
        q, k, (((1,), (1,)), ((), ())), preferred_element_type=jnp.float32)
    q2 = jnp.sum(q * q, axis=1, keepdims=True)
    mk2 = jnp.sum(k * k, axis=1)[None, :]
    d2_ref[0] = (q2 - 2.0 * qk) + mk2


def _distances(q, mem_keys, bt=512, ct=2048):
    A, C, D = mem_keys.shape
    B = q.shape[0]
    grid = (A, B // bt, pl.cdiv(C, ct))
    return pl.pallas_call(
        _dist_body,
        grid=grid,
        in_specs=[
            pl.BlockSpec((bt, D), lambda a, b, c: (b, 0)),
            pl.BlockSpec((1, ct, D), lambda a, b, c: (a, c, 0)),
        ],
        out_specs=pl.BlockSpec((1, bt, ct), lambda a, b, c: (a, b, c)),
        out_shape=jax.ShapeDtypeStruct((A, B, C), jnp.float32),
    )(q, mem_keys)


def kernel(x, W1, b1, W2, b2, mem_keys, mem_values):
    key = _encoder(x, W1, b1, W2, b2)
    d2 = _distances(key, mem_keys)
    top_neg, top_idx = jax.lax.top_k(-d2, K_NEIGHBORS)
    dists = -top_neg
    w = 1.0 / (dists + DELTA)
    w = w / jnp.sum(w, axis=-1, keepdims=True)
    v = jax.vmap(lambda mv, ti: mv[ti])(mem_values, top_idx)
    q_vals = jnp.sum(w * v, axis=-1)
    values = q_vals.T
    action = jnp.argmax(values, axis=1)
    indexes = jnp.transpose(top_idx, (1, 0, 2))
    scores = jnp.transpose(w, (1, 0, 2))
    return (key, values, action, indexes, scores)


# speedup vs baseline: 57.2518x; 57.2518x over previous
import functools

import jax
import jax.numpy as jnp
from jax.experimental import pallas as pl
from jax.experimental.pallas import tpu as pltpu

K_NEIGHBORS = 50
DELTA = 1e-3
NEG_INF = float('-inf')
M_SLOTS = 10
LANES = 128


def _enc_body(x_ref, w1_ref, b1_ref, w2_ref, b2_ref, key_ref):
    h = jax.lax.dot_general(
        x_ref[...], w1_ref[...], (((1,), (0,)), ((), ())),
        preferred_element_type=jnp.float32)
    h = jnp.maximum(h + b1_ref[...], 0.0)
    key_ref[...] = jax.lax.dot_general(
        h, w2_ref[...], (((1,), (0,)), ((), ())),
        preferred_element_type=jnp.float32) + b2_ref[...]


def _encoder(x, W1, b1, W2, b2):
    B = x.shape[0]
    D = W2.shape[1]
    return pl.pallas_call(
        _enc_body,
        out_shape=jax.ShapeDtypeStruct((B, D), jnp.float32),
    )(x, W1, b1.reshape(1, -1), W2, b2.reshape(1, -1))


def _topk_body(q_ref, k_ref, vals_ref, idx_ref, kept_v, kept_i,
               cand_v_ref, cand_i_ref, *, ct, C, k, m):
    c = pl.program_id(2)
    nc = pl.num_programs(2)
    bt = q_ref.shape[0]

    @pl.when(c == 0)
    def _init():
        kept_v[...] = jnp.full_like(kept_v, NEG_INF)
        kept_i[...] = jnp.zeros_like(kept_i)

    q = q_ref[...]
    kk = k_ref[0]
    qk = jax.lax.dot_general(
        q, kk, (((1,), (1,)), ((), ())), preferred_element_type=jnp.float32)
    q2 = jnp.sum(q * q, axis=1, keepdims=True)
    mk2 = jnp.sum(kk * kk, axis=1)[None, :]
    neg = (2.0 * qk - q2) - mk2

    col = c * ct + jax.lax.broadcasted_iota(jnp.int32, neg.shape, 1)
    neg = jnp.where(col < C, neg, NEG_INF)

    for h in range(ct // LANES):
        x_v = neg[:, h * LANES:(h + 1) * LANES]
        x_i = (c * ct + h * LANES
               + jax.lax.broadcasted_iota(jnp.int32, (bt, LANES), 1))
        for j in range(m):
            kv = kept_v[j]
            ki = kept_i[j]
            take = x_v > kv
            kept_v[j] = jnp.where(take, x_v, kv)
            kept_i[j] = jnp.where(take, x_i, ki)
            x_v = jnp.where(take, kv, x_v)
            x_i = jnp.where(take, ki, x_i)

    @pl.when(c == nc - 1)
    def _emit():
        cand_v_ref[...] = jnp.concatenate(
            [kept_v[j] for j in range(m)], axis=1)
        cand_i_ref[...] = jnp.concatenate(
            [kept_i[j] for j in range(m)], axis=1)
        W = m * LANES
        pos = jax.lax.broadcasted_iota(jnp.int32, (bt, W), 1)
        kw = vals_ref.shape[2]
        slot = jax.lax.broadcasted_iota(jnp.int32, (bt, kw), 1)

        def body(t, _):
            cv = cand_v_ref[...]
            cur = jnp.max(cv, axis=1, keepdims=True)
            am = jnp.argmax(cv, axis=1)
            hot = pos == am[:, None]
            idx = jnp.sum(jnp.where(hot, cand_i_ref[...], 0),
                          axis=1, keepdims=True)
            cand_v_ref[...] = jnp.where(hot, NEG_INF, cv)
            sel = slot == t
            vals_ref[0] = jnp.where(sel, cur, vals_ref[0])
            idx_ref[0] = jnp.where(sel, idx, idx_ref[0])
            return 0

        jax.lax.fori_loop(0, k, body, 0)


def _fused_topk(q, mem_keys, bt=256, ct=2048, k=K_NEIGHBORS, m=M_SLOTS):
    A, C, D = mem_keys.shape
    B = q.shape[0]
    nc = pl.cdiv(C, ct)
    grid = (A, B // bt, nc)
    kw = 64
    body = functools.partial(_topk_body, ct=ct, C=C, k=k, m=m)
    vals, idx = pl.pallas_call(
        body,
        grid=grid,
        in_specs=[
            pl.BlockSpec((bt, D), lambda a, b, c: (b, 0)),
            pl.BlockSpec((1, ct, D), lambda a, b, c: (a, c, 0)),
        ],
        out_specs=[
            pl.BlockSpec((1, bt, kw), lambda a, b, c: (a, b, 0)),
            pl.BlockSpec((1, bt, kw), lambda a, b, c: (a, b, 0)),
        ],
        out_shape=[
            jax.ShapeDtypeStruct((A, B, kw), jnp.float32),
            jax.ShapeDtypeStruct((A, B, kw), jnp.int32),
        ],
        scratch_shapes=[
            pltpu.VMEM((m, bt, LANES), jnp.float32),
            pltpu.VMEM((m, bt, LANES), jnp.int32),
            pltpu.VMEM((bt, m * LANES), jnp.float32),
            pltpu.VMEM((bt, m * LANES), jnp.int32),
        ],
    )(q, mem_keys)
    return vals[:, :, :k], idx[:, :, :k]


def kernel(x, W1, b1, W2, b2, mem_keys, mem_values):
    key = _encoder(x, W1, b1, W2, b2)
    top_neg, top_idx = _fused_topk(key, mem_keys)
    dists = -top_neg
    w = 1.0 / (dists + DELTA)
    w = w / jnp.sum(w, axis=-1, keepdims=True)
    v = jax.vmap(lambda mv, ti: mv[ti])(mem_values, top_idx)
    q_vals = jnp.sum(w * v, axis=-1)
    values = q_vals.T
    action = jnp.argmax(values, axis=1)
    indexes = jnp.transpose(top_idx, (1, 0, 2))
    scores = jnp.transpose(w, (1, 0, 2))
    return (key, values, action, indexes, scores)


# speedup vs baseline: 57.6462x; 1.0069x over previous
import functools

import jax
import jax.numpy as jnp
from jax.experimental import pallas as pl
from jax.experimental.pallas import tpu as pltpu

K_NEIGHBORS = 50
DELTA = 1e-3
NEG_INF = float('-inf')
M_SLOTS = 10
LANES = 128


def _enc_body(x_ref, w1_ref, b1_ref, w2_ref, b2_ref, key_ref):
    h = jax.lax.dot_general(
        x_ref[...], w1_ref[...], (((1,), (0,)), ((), ())),
        preferred_element_type=jnp.float32)
    h = jnp.maximum(h + b1_ref[...], 0.0)
    key_ref[...] = jax.lax.dot_general(
        h, w2_ref[...], (((1,), (0,)), ((), ())),
        preferred_element_type=jnp.float32) + b2_ref[...]


def _encoder(x, W1, b1, W2, b2):
    B = x.shape[0]
    D = W2.shape[1]
    return pl.pallas_call(
        _enc_body,
        out_shape=jax.ShapeDtypeStruct((B, D), jnp.float32),
    )(x, W1, b1.reshape(1, -1), W2, b2.reshape(1, -1))


def _topk_body(q_ref, k_ref, vals_ref, idx_ref, kept_v, kept_i,
               cand_v_ref, cand_i_ref, *, ct, C, k, m):
    c = pl.program_id(2)
    nc = pl.num_programs(2)
    bt = q_ref.shape[0]

    @pl.when(c == 0)
    def _init():
        kept_v[...] = jnp.full_like(kept_v, NEG_INF)
        kept_i[...] = jnp.zeros_like(kept_i)

    q = q_ref[...]
    kk = k_ref[0]
    qk = jax.lax.dot_general(
        q, kk, (((1,), (1,)), ((), ())), preferred_element_type=jnp.float32)
    q2 = jnp.sum(q * q, axis=1, keepdims=True)
    mk2 = jnp.sum(kk * kk, axis=1)[None, :]
    neg = (2.0 * qk - q2) - mk2

    col = c * ct + jax.lax.broadcasted_iota(jnp.int32, neg.shape, 1)
    neg = jnp.where(col < C, neg, NEG_INF)

    for h in range(ct // LANES):
        x_v = neg[:, h * LANES:(h + 1) * LANES]
        x_i = (c * ct + h * LANES
               + jax.lax.broadcasted_iota(jnp.int32, (bt, LANES), 1))
        for j in range(m):
            kv = kept_v[j]
            ki = kept_i[j]
            take = x_v > kv
            kept_v[j] = jnp.where(take, x_v, kv)
            kept_i[j] = jnp.where(take, x_i, ki)
            x_v = jnp.where(take, kv, x_v)
            x_i = jnp.where(take, ki, x_i)

    @pl.when(c == nc - 1)
    def _emit():
        cand_v_ref[...] = jnp.concatenate(
            [kept_v[j] for j in range(m)], axis=1)
        cand_i_ref[...] = jnp.concatenate(
            [kept_i[j] for j in range(m)], axis=1)
        W = m * LANES
        pos = jax.lax.broadcasted_iota(jnp.int32, (bt, W), 1)
        kw = vals_ref.shape[2]
        slot = jax.lax.broadcasted_iota(jnp.int32, (bt, kw), 1)

        def body(t, _):
            cv = cand_v_ref[...]
            cur = jnp.max(cv, axis=1, keepdims=True)
            am = jnp.argmax(cv, axis=1)
            hot = pos == am[:, None]
            idx = jnp.sum(jnp.where(hot, cand_i_ref[...], 0),
                          axis=1, keepdims=True)
            cand_v_ref[...] = jnp.where(hot, NEG_INF, cv)
            sel = slot == t
            vals_ref[0] = jnp.where(sel, cur, vals_ref[0])
            idx_ref[0] = jnp.where(sel, idx, idx_ref[0])
            return 0

        jax.lax.fori_loop(0, k, body, 0)


def _fused_topk(q, mem_keys, bt=256, ct=2048, k=K_NEIGHBORS, m=M_SLOTS):
    A, C, D = mem_keys.shape
    B = q.shape[0]
    nc = pl.cdiv(C, ct)
    grid = (A, B // bt, nc)
    kw = 64
    body = functools.partial(_topk_body, ct=ct, C=C, k=k, m=m)
    vals, idx = pl.pallas_call(
        body,
        grid=grid,
        in_specs=[
            pl.BlockSpec((bt, D), lambda a, b, c: (b, 0)),
            pl.BlockSpec((1, ct, D), lambda a, b, c: (a, c, 0)),
        ],
        out_specs=[
            pl.BlockSpec((1, bt, kw), lambda a, b, c: (a, b, 0)),
            pl.BlockSpec((1, bt, kw), lambda a, b, c: (a, b, 0)),
        ],
        out_shape=[
            jax.ShapeDtypeStruct((A, B, kw), jnp.float32),
            jax.ShapeDtypeStruct((A, B, kw), jnp.int32),
        ],
        scratch_shapes=[
            pltpu.VMEM((m, bt, LANES), jnp.float32),
            pltpu.VMEM((m, bt, LANES), jnp.int32),
            pltpu.VMEM((bt, m * LANES), jnp.float32),
            pltpu.VMEM((bt, m * LANES), jnp.int32),
        ],
        compiler_params=pltpu.CompilerParams(
            dimension_semantics=("parallel", "parallel", "arbitrary")),
    )(q, mem_keys)
    return vals[:, :, :k], idx[:, :, :k]


def kernel(x, W1, b1, W2, b2, mem_keys, mem_values):
    key = _encoder(x, W1, b1, W2, b2)
    top_neg, top_idx = _fused_topk(key, mem_keys)
    dists = -top_neg
    w = 1.0 / (dists + DELTA)
    w = w / jnp.sum(w, axis=-1, keepdims=True)
    v = jax.vmap(lambda mv, ti: mv[ti])(mem_values, top_idx)
    q_vals = jnp.sum(w * v, axis=-1)
    values = q_vals.T
    action = jnp.argmax(values, axis=1)
    indexes = jnp.transpose(top_idx, (1, 0, 2))
    scores = jnp.transpose(w, (1, 0, 2))
    return (key, values, action, indexes, scores)
